# baseline (speedup 1.0000x reference)
import jax
import jax.numpy as jnp
from jax.experimental import pallas as pl
from jax.experimental.pallas import tpu as pltpu

_STRIP = 512


def _make_strip_kernel(n):
    nk = -(-n // _STRIP)
    valid_last = n - (nk - 1) * _STRIP

    def _gcn_strip_kernel(adj_ref, x_ref, w_ref, acc_ref):
        k = pl.program_id(0)

        @pl.when(k == nk - 1)
        def _zero_pad_cols():
            adj_ref[:, valid_last:] = jnp.zeros(
                (n, _STRIP - valid_last), jnp.float32)

        strip = adj_ref[...]
        colsum = jnp.sum(strip, axis=0, keepdims=True)
        col_ok = (k * _STRIP
                  + jax.lax.broadcasted_iota(jnp.int32, colsum.shape, 1)) < n
        colsum = jnp.where(col_ok, colsum, 1.0)
        s = jax.lax.rsqrt(jnp.clip(colsum, 1e-6, None))[0]
        xa = x_ref[...]
        hk = (xa[:, :-1] * s[:, None]) @ w_ref[...]
        hk_aug = jnp.concatenate([hk, xa[:, -1:]], axis=1)
        h = n // 2
        hk_bf = hk_aug.astype(jnp.bfloat16)
        top = jnp.dot(strip[:h].astype(jnp.bfloat16), hk_bf,
                      preferred_element_type=jnp.float32)
        bot = jnp.dot(strip[h:].astype(jnp.bfloat16), hk_bf,
                      preferred_element_type=jnp.float32)
        prev_t = jnp.where(k > 0, acc_ref[:h, :], 0.0)
        acc_ref[:h, :] = prev_t + top
        prev_b = jnp.where(k > 0, acc_ref[h:, :], 0.0)
        acc_ref[h:, :] = prev_b + bot

    return _gcn_strip_kernel


def _epilogue_kernel(acc_ref, b_ref, out_ref):
    acc = acc_ref[...]
    nd = jax.lax.rsqrt(jnp.clip(acc[:, -1:], 1e-6, None))
    out_ref[...] = jnp.maximum(acc[:, :-1] * nd + b_ref[...], 0.0)


def kernel(adj, x, W, b):
    n, _ = adj.shape
    d_in = x.shape[1]
    d_out = W.shape[1]
    nk = -(-n // _STRIP)
    n_pad = nk * _STRIP
    xa = jnp.concatenate(
        [x, jnp.ones((n, 1), x.dtype)], axis=1)
    xa = jnp.pad(xa, ((0, n_pad - n), (0, 0)))

    acc = pl.pallas_call(
        _make_strip_kernel(n),
        grid=(nk,),
        in_specs=[
            pl.BlockSpec((n, _STRIP), lambda k: (0, k)),
            pl.BlockSpec((_STRIP, d_in + 1), lambda k: (k, 0)),
            pl.BlockSpec((d_in, d_out), lambda k: (0, 0)),
        ],
        out_specs=pl.BlockSpec((n, d_out + 1), lambda k: (0, 0)),
        out_shape=jax.ShapeDtypeStruct((n, d_out + 1), jnp.float32),
        compiler_params=pltpu.CompilerParams(
            dimension_semantics=("arbitrary",),
            vmem_limit_bytes=100 * 1024 * 1024,
        ),
    )(adj, xa, W)

    return pl.pallas_call(
        _epilogue_kernel,
        in_specs=[
            pl.BlockSpec((n, d_out + 1), lambda: (0, 0)),
            pl.BlockSpec((1, d_out), lambda: (0, 0)),
        ],
        out_specs=pl.BlockSpec((n, d_out), lambda: (0, 0)),
        out_shape=jax.ShapeDtypeStruct((n, d_out), jnp.float32),
    )(acc, b.reshape(1, d_out))


# speedup vs baseline: 1.0247x; 1.0247x over previous
import jax
import jax.numpy as jnp
from jax.experimental import pallas as pl
from jax.experimental.pallas import tpu as pltpu

_STRIP = 512


def _make_gcn_kernel(n, valid_last):
    def _gcn_strip_kernel(adj_ref, x_ref, w_ref, b_ref, out_ref, rowsum_ref):
        k = pl.program_id(0)
        nk = pl.num_programs(0)

        @pl.when(k == nk - 1)
        def _zero_pad_cols():
            adj_ref[:, valid_last:] = jnp.zeros(
                (n, _STRIP - valid_last), jnp.float32)

        strip = adj_ref[...]
        colsum = jnp.sum(strip, axis=0)
        s = jax.lax.rsqrt(jnp.clip(colsum, 1e-6, None))
        hk = (x_ref[...] * s[:, None]) @ w_ref[...]
        partial = jnp.dot(strip, hk, preferred_element_type=jnp.float32)
        rs = jnp.sum(strip, axis=1, keepdims=True)

        @pl.when(k == 0)
        def _init():
            out_ref[...] = partial
            rowsum_ref[...] = rs

        @pl.when(k > 0)
        def _accum():
            out_ref[...] += partial
            rowsum_ref[...] += rs

        @pl.when(k == nk - 1)
        def _epilogue():
            nd = jax.lax.rsqrt(jnp.clip(rowsum_ref[...], 1e-6, None))
            out_ref[...] = jnp.maximum(out_ref[...] * nd + b_ref[...], 0.0)

    return _gcn_strip_kernel


def kernel(adj, x, W, b):
    n, _ = adj.shape
    d_in = x.shape[1]
    d_out = W.shape[1]
    nk = -(-n // _STRIP)
    valid_last = n - (nk - 1) * _STRIP
    x_pad = jnp.pad(x, ((0, nk * _STRIP - n), (0, 0)))

    return pl.pallas_call(
        _make_gcn_kernel(n, valid_last),
        grid=(nk,),
        in_specs=[
            pl.BlockSpec((n, _STRIP), lambda k: (0, k)),
            pl.BlockSpec((_STRIP, d_in), lambda k: (k, 0)),
            pl.BlockSpec((d_in, d_out), lambda k: (0, 0)),
            pl.BlockSpec((1, d_out), lambda k: (0, 0)),
        ],
        out_specs=pl.BlockSpec((n, d_out), lambda k: (0, 0)),
        out_shape=jax.ShapeDtypeStruct((n, d_out), jnp.float32),
        scratch_shapes=[pltpu.VMEM((n, 1), jnp.float32)],
        compiler_params=pltpu.CompilerParams(
            dimension_semantics=("arbitrary",),
            vmem_limit_bytes=110 * 1024 * 1024,
        ),
    )(adj, x_pad, W, b.reshape(1, d_out))


# speedup vs baseline: 1.0361x; 1.0111x over previous
import jax
import jax.numpy as jnp
from jax.experimental import pallas as pl
from jax.experimental.pallas import tpu as pltpu

_STRIP = 512
_CHUNK = 2000


def _make_gcn_kernel(n, valid_last):
    def _gcn_strip_kernel(adj_ref, x_ref, w_ref, b_ref, out_ref, rowsum_ref):
        k = pl.program_id(0)
        nk = pl.num_programs(0)

        @pl.when(k == nk - 1)
        def _zero_pad_cols():
            adj_ref[:, valid_last:] = jnp.zeros(
                (n, _STRIP - valid_last), jnp.float32)

        @pl.when(k == 0)
        def _init():
            out_ref[...] = jnp.zeros((n, out_ref.shape[1]), jnp.float32)
            rowsum_ref[...] = jnp.zeros((n, 1), jnp.float32)

        colsum = jnp.zeros((_STRIP,), jnp.float32)
        for r in range(0, n, _CHUNK):
            colsum = colsum + jnp.sum(adj_ref[r:r + _CHUNK, :], axis=0)
        s = jax.lax.rsqrt(jnp.clip(colsum, 1e-6, None))
        hk = (x_ref[...] * s[:, None]) @ w_ref[...]
        hk_b = hk.astype(jnp.bfloat16)

        for r in range(0, n, _CHUNK):
            chunk = adj_ref[r:r + _CHUNK, :]
            out_ref[r:r + _CHUNK, :] += jnp.dot(
                chunk.astype(jnp.bfloat16), hk_b,
                preferred_element_type=jnp.float32)
            rowsum_ref[r:r + _CHUNK, :] += jnp.sum(
                chunk, axis=1, keepdims=True)

        @pl.when(k == nk - 1)
        def _epilogue():
            for r in range(0, n, _CHUNK):
                nd = jax.lax.rsqrt(
                    jnp.clip(rowsum_ref[r:r + _CHUNK, :], 1e-6, None))
                out_ref[r:r + _CHUNK, :] = jnp.maximum(
                    out_ref[r:r + _CHUNK, :] * nd + b_ref[...], 0.0)

    return _gcn_strip_kernel


def kernel(adj, x, W, b):
    n, _ = adj.shape
    d_in = x.shape[1]
    d_out = W.shape[1]
    nk = -(-n // _STRIP)
    valid_last = n - (nk - 1) * _STRIP
    x_pad = jnp.pad(x, ((0, nk * _STRIP - n), (0, 0)))

    return pl.pallas_call(
        _make_gcn_kernel(n, valid_last),
        grid=(nk,),
        in_specs=[
            pl.BlockSpec((n, _STRIP), lambda k: (0, k)),
            pl.BlockSpec((_STRIP, d_in), lambda k: (k, 0)),
            pl.BlockSpec((d_in, d_out), lambda k: (0, 0)),
            pl.BlockSpec((1, d_out), lambda k: (0, 0)),
        ],
        out_specs=pl.BlockSpec((n, d_out), lambda k: (0, 0)),
        out_shape=jax.ShapeDtypeStruct((n, d_out), jnp.float32),
        scratch_shapes=[pltpu.VMEM((n, 1), jnp.float32)],
        compiler_params=pltpu.CompilerParams(
            dimension_semantics=("arbitrary",),
            vmem_limit_bytes=110 * 1024 * 1024,
        ),
    )(adj, x_pad, W, b.reshape(1, d_out))
